# baseline (speedup 1.0000x reference)
import jax
import jax.numpy as jnp
from jax.experimental import pallas as pl

_TN = 40


def _fused_body(nl_ref, rbf_ref, mask_ref, feat_ref, w1_ref, b1_ref,
                w2_ref, b2_ref, out_ref):
    rows, g_dim = rbf_ref.shape[1], rbf_ref.shape[2]
    n = feat_ref.shape[1]
    f = feat_ref.shape[2]
    k = rows // out_ref.shape[1]

    rbf = rbf_ref[0]
    h = jnp.dot(rbf, w1_ref[...], preferred_element_type=jnp.float32)
    h = h + b1_ref[...]
    h = jax.nn.softplus(h) - jnp.log(2.0)
    filt = jnp.dot(h.astype(jnp.bfloat16), w2_ref[...],
                   preferred_element_type=jnp.float32)
    filt = (filt + b2_ref[...]) * mask_ref[0]

    nl = nl_ref[0]
    lane = jax.lax.broadcasted_iota(jnp.int32, (1, n), 1)
    onehot = (nl == lane).astype(jnp.bfloat16)
    gathered = jnp.dot(onehot, feat_ref[0],
                       preferred_element_type=jnp.float32)

    prod = filt * gathered
    out_ref[0] = prod.reshape(out_ref.shape[1], k, f).sum(axis=1)


def kernel(features, rbf_expansion, neighbor_list, neighbor_mask,
           W1, b1, W2, b2):
    B, N, F = features.shape
    _, _, K, G = rbf_expansion.shape
    tn = _TN
    rows = tn * K

    feat_bf = features.astype(jnp.bfloat16)
    nl = neighbor_list.reshape(B, N * K, 1)
    mask = neighbor_mask.reshape(B, N * K, 1)
    rbf = rbf_expansion.reshape(B, N * K, G).astype(jnp.bfloat16)
    b1r = b1.reshape(1, F)
    b2r = b2.reshape(1, F)
    w1 = W1.astype(jnp.bfloat16)
    w2 = W2.astype(jnp.bfloat16)

    return pl.pallas_call(
        _fused_body,
        grid=(B, N // tn),
        in_specs=[
            pl.BlockSpec((1, rows, 1), lambda b, t: (b, t, 0)),
            pl.BlockSpec((1, rows, G), lambda b, t: (b, t, 0)),
            pl.BlockSpec((1, rows, 1), lambda b, t: (b, t, 0)),
            pl.BlockSpec((1, N, F), lambda b, t: (b, 0, 0)),
            pl.BlockSpec((G, F), lambda b, t: (0, 0)),
            pl.BlockSpec((1, F), lambda b, t: (0, 0)),
            pl.BlockSpec((F, F), lambda b, t: (0, 0)),
            pl.BlockSpec((1, F), lambda b, t: (0, 0)),
        ],
        out_specs=pl.BlockSpec((1, tn, F), lambda b, t: (b, t, 0)),
        out_shape=jax.ShapeDtypeStruct((B, N, F), jnp.float32),
    )(nl, rbf, mask, feat_bf, w1, b1r, w2, b2r)


# speedup vs baseline: 1.5159x; 1.5159x over previous
import jax
import jax.numpy as jnp
from jax.experimental import pallas as pl

_TN = 40


def _fused_body(nl_ref, rbf_ref, mask_ref, feat_ref, w1_ref, b1_ref,
                w2_ref, b2_ref, out_ref):
    tn, k, g_dim = rbf_ref.shape[1], rbf_ref.shape[2], rbf_ref.shape[3]
    n = feat_ref.shape[1]
    f = feat_ref.shape[2]
    rows = tn * k

    rbf = rbf_ref[0].reshape(rows, g_dim).astype(jnp.bfloat16)
    h = jnp.dot(rbf, w1_ref[...], preferred_element_type=jnp.float32)
    h = h + b1_ref[...]
    h = jax.nn.softplus(h) - jnp.log(2.0)
    filt = jnp.dot(h.astype(jnp.bfloat16), w2_ref[...],
                   preferred_element_type=jnp.float32)
    filt = filt + b2_ref[...]

    nl3 = jax.lax.broadcast_in_dim(nl_ref[0], (tn, k, n), (0, 1))
    mask3 = jax.lax.broadcast_in_dim(mask_ref[0], (tn, k, n), (0, 1))
    bead = jax.lax.broadcasted_iota(jnp.int32, (tn, k, n), 2)
    onehot = jnp.where(nl3 == bead, mask3, 0.0).astype(jnp.bfloat16)
    gathered = jnp.dot(onehot.reshape(rows, n),
                       feat_ref[0].astype(jnp.bfloat16),
                       preferred_element_type=jnp.float32)

    prod = filt * gathered
    out_ref[0] = prod.reshape(tn, k, f).sum(axis=1)


def kernel(features, rbf_expansion, neighbor_list, neighbor_mask,
           W1, b1, W2, b2):
    B, N, F = features.shape
    _, _, K, G = rbf_expansion.shape
    tn = _TN

    b1r = b1.reshape(1, F)
    b2r = b2.reshape(1, F)
    w1 = W1.astype(jnp.bfloat16)
    w2 = W2.astype(jnp.bfloat16)

    return pl.pallas_call(
        _fused_body,
        grid=(B, N // tn),
        in_specs=[
            pl.BlockSpec((1, tn, K), lambda b, t: (b, t, 0)),
            pl.BlockSpec((1, tn, K, G), lambda b, t: (b, t, 0, 0)),
            pl.BlockSpec((1, tn, K), lambda b, t: (b, t, 0)),
            pl.BlockSpec((1, N, F), lambda b, t: (b, 0, 0)),
            pl.BlockSpec((G, F), lambda b, t: (0, 0)),
            pl.BlockSpec((1, F), lambda b, t: (0, 0)),
            pl.BlockSpec((F, F), lambda b, t: (0, 0)),
            pl.BlockSpec((1, F), lambda b, t: (0, 0)),
        ],
        out_specs=pl.BlockSpec((1, tn, F), lambda b, t: (b, t, 0)),
        out_shape=jax.ShapeDtypeStruct((B, N, F), jnp.float32),
    )(neighbor_list, rbf_expansion, neighbor_mask, features, w1, b1r, w2, b2r)


# speedup vs baseline: 1.7431x; 1.1499x over previous
import jax
import jax.numpy as jnp
from jax.experimental import pallas as pl

_TN = 200


def _fused_body(nl_ref, rbf_ref, mask_ref, feat_ref, w1_ref, b1_ref,
                w2_ref, b2_ref, out_ref):
    tn, k, g_dim = rbf_ref.shape[1], rbf_ref.shape[2], rbf_ref.shape[3]
    n = feat_ref.shape[1]
    f = feat_ref.shape[2]
    rows = tn * k

    rbf = rbf_ref[0].reshape(rows, g_dim).astype(jnp.bfloat16)
    h = jnp.dot(rbf, w1_ref[...], preferred_element_type=jnp.float32)
    h = h + b1_ref[...]
    h = jax.nn.softplus(h) - jnp.log(2.0)
    filt = jnp.dot(h.astype(jnp.bfloat16), w2_ref[...],
                   preferred_element_type=jnp.float32)
    filt = filt + b2_ref[...]

    nl3 = jax.lax.broadcast_in_dim(nl_ref[0], (tn, k, n), (0, 1))
    mask3 = jax.lax.broadcast_in_dim(mask_ref[0], (tn, k, n), (0, 1))
    bead = jax.lax.broadcasted_iota(jnp.int32, (tn, k, n), 2)
    onehot = jnp.where(nl3 == bead, mask3, 0.0).astype(jnp.bfloat16)
    gathered = jnp.dot(onehot.reshape(rows, n),
                       feat_ref[0].astype(jnp.bfloat16),
                       preferred_element_type=jnp.float32)

    prod = filt * gathered
    out_ref[0] = prod.reshape(tn, k, f).sum(axis=1)


def kernel(features, rbf_expansion, neighbor_list, neighbor_mask,
           W1, b1, W2, b2):
    B, N, F = features.shape
    _, _, K, G = rbf_expansion.shape
    tn = _TN

    b1r = b1.reshape(1, F)
    b2r = b2.reshape(1, F)
    w1 = W1.astype(jnp.bfloat16)
    w2 = W2.astype(jnp.bfloat16)

    return pl.pallas_call(
        _fused_body,
        grid=(B, N // tn),
        in_specs=[
            pl.BlockSpec((1, tn, K), lambda b, t: (b, t, 0)),
            pl.BlockSpec((1, tn, K, G), lambda b, t: (b, t, 0, 0)),
            pl.BlockSpec((1, tn, K), lambda b, t: (b, t, 0)),
            pl.BlockSpec((1, N, F), lambda b, t: (b, 0, 0)),
            pl.BlockSpec((G, F), lambda b, t: (0, 0)),
            pl.BlockSpec((1, F), lambda b, t: (0, 0)),
            pl.BlockSpec((F, F), lambda b, t: (0, 0)),
            pl.BlockSpec((1, F), lambda b, t: (0, 0)),
        ],
        out_specs=pl.BlockSpec((1, tn, F), lambda b, t: (b, t, 0)),
        out_shape=jax.ShapeDtypeStruct((B, N, F), jnp.float32),
    )(neighbor_list, rbf_expansion, neighbor_mask, features, w1, b1r, w2, b2r)
